# baseline (speedup 1.0000x reference)
import jax
import jax.numpy as jnp
from jax.experimental import pallas as pl
from jax.experimental.pallas import tpu as pltpu

PATCH = 8
VDIM = 8
OUT_DIM = 768
NSIDE = 12
NPATCH = NSIDE ** 3
VPP = PATCH ** 3
SLAB = NSIDE * NSIDE


def _fused_kernel(x_ref, s_ref, u_ref, b_ref, out_ref):
    dn = (((1,), (1,)), ((), ()))
    for nd in range(NSIDE):
        t = x_ref[0, nd * PATCH:(nd + 1) * PATCH]
        xp = t.reshape(PATCH, NSIDE, PATCH, NSIDE, PATCH)
        xp = xp.transpose(1, 3, 0, 2, 4).reshape(SLAB, VPP)
        m = (xp >= -1000.0) & (xp < -75.0)
        a = jnp.where(m, xp, 0.0).astype(jnp.bfloat16)
        nb = jnp.where(m, 0.0, 1.0).astype(jnp.bfloat16)
        acc = jax.lax.dot_general(s_ref[...], a, dn,
                                  preferred_element_type=jnp.float32)
        acc += jax.lax.dot_general(u_ref[...], nb, dn,
                                   preferred_element_type=jnp.float32)
        out_ref[0, :, nd * SLAB:(nd + 1) * SLAB] = acc + b_ref[...]


def kernel(x, vectors, cls_vectors, fc_w, fc_b):
    B = x.shape[0]
    xs = x.reshape(B, 96, 96, 96)
    w3 = fc_w.reshape(OUT_DIM, VPP, VDIM)
    s = w3.sum(-1).astype(jnp.bfloat16)
    uv = (w3 @ vectors[9]).astype(jnp.bfloat16)
    b2 = fc_b.reshape(OUT_DIM, 1)

    out = pl.pallas_call(
        _fused_kernel,
        grid=(B,),
        in_specs=[
            pl.BlockSpec((1, 96, 96, 96), lambda b: (b, 0, 0, 0)),
            pl.BlockSpec((OUT_DIM, VPP), lambda b: (0, 0)),
            pl.BlockSpec((OUT_DIM, VPP), lambda b: (0, 0)),
            pl.BlockSpec((OUT_DIM, 1), lambda b: (0, 0)),
        ],
        out_specs=pl.BlockSpec((1, OUT_DIM, NPATCH), lambda b: (b, 0, 0)),
        out_shape=jax.ShapeDtypeStruct((B, OUT_DIM, NPATCH), jnp.float32),
        compiler_params=pltpu.CompilerParams(
            dimension_semantics=("parallel",),
        ),
    )(xs, s, uv, b2)

    return out.reshape(B, OUT_DIM, NSIDE, NSIDE, NSIDE)


# speedup vs baseline: 1.3798x; 1.3798x over previous
import jax
import jax.numpy as jnp
from jax.experimental import pallas as pl
from jax.experimental.pallas import tpu as pltpu

PATCH = 8
VDIM = 8
OUT_DIM = 768
NSIDE = 12
NPATCH = NSIDE ** 3
VPP = PATCH ** 3
SLAB = NSIDE * NSIDE


def _pack_kernel(x_ref, a_ref, n_ref):
    t = x_ref[0]
    xp = t.reshape(PATCH, NSIDE, PATCH, NSIDE, PATCH)
    xp = xp.transpose(1, 3, 0, 2, 4).reshape(SLAB, VPP)
    m = (xp >= -1000.0) & (xp < -75.0)
    a_ref[0] = jnp.where(m, xp, 0.0).astype(jnp.bfloat16)
    n_ref[0] = jnp.where(m, 0.0, 1.0).astype(jnp.bfloat16)


def _gemm_kernel(a_ref, n_ref, s_ref, u_ref, b_ref, out_ref):
    dn = (((1,), (1,)), ((), ()))
    acc = jax.lax.dot_general(s_ref[...], a_ref[0], dn,
                              preferred_element_type=jnp.float32)
    acc += jax.lax.dot_general(u_ref[...], n_ref[0], dn,
                               preferred_element_type=jnp.float32)
    out_ref[0] = acc + b_ref[...]


def kernel(x, vectors, cls_vectors, fc_w, fc_b):
    B = x.shape[0]
    xs = x.reshape(B, 96, 96, 96)
    w3 = fc_w.reshape(OUT_DIM, VPP, VDIM)
    s = w3.sum(-1).astype(jnp.bfloat16)
    uv = (w3 @ vectors[9]).astype(jnp.bfloat16)
    b2 = fc_b.reshape(OUT_DIM, 1)

    a, nb = pl.pallas_call(
        _pack_kernel,
        grid=(B, NSIDE),
        in_specs=[
            pl.BlockSpec((1, PATCH, 96, 96), lambda b, nd: (b, nd, 0, 0)),
        ],
        out_specs=[
            pl.BlockSpec((1, SLAB, VPP), lambda b, nd: (b, nd, 0)),
            pl.BlockSpec((1, SLAB, VPP), lambda b, nd: (b, nd, 0)),
        ],
        out_shape=[
            jax.ShapeDtypeStruct((B, NPATCH, VPP), jnp.bfloat16),
            jax.ShapeDtypeStruct((B, NPATCH, VPP), jnp.bfloat16),
        ],
        compiler_params=pltpu.CompilerParams(
            dimension_semantics=("parallel", "parallel"),
        ),
    )(xs)

    out = pl.pallas_call(
        _gemm_kernel,
        grid=(B,),
        in_specs=[
            pl.BlockSpec((1, NPATCH, VPP), lambda b: (b, 0, 0)),
            pl.BlockSpec((1, NPATCH, VPP), lambda b: (b, 0, 0)),
            pl.BlockSpec((OUT_DIM, VPP), lambda b: (0, 0)),
            pl.BlockSpec((OUT_DIM, VPP), lambda b: (0, 0)),
            pl.BlockSpec((OUT_DIM, 1), lambda b: (0, 0)),
        ],
        out_specs=pl.BlockSpec((1, OUT_DIM, NPATCH), lambda b: (b, 0, 0)),
        out_shape=jax.ShapeDtypeStruct((B, OUT_DIM, NPATCH), jnp.float32),
        compiler_params=pltpu.CompilerParams(
            dimension_semantics=("parallel",),
        ),
    )(a, nb, s, uv, b2)

    return out.reshape(B, OUT_DIM, NSIDE, NSIDE, NSIDE)
